# baseline (speedup 1.0000x reference)
import functools

import jax
import jax.numpy as jnp
from jax import lax
from jax.experimental import pallas as pl
from jax.experimental.pallas import tpu as pltpu
from jax.experimental.pallas import tpu_sc as plsc

N = 10000
M = 12
AF = 64
NBR_F = 16
F2 = 2 * AF
NCRYS = 128
NM = N * M

_NC = 2
_NS = 16
_NW = _NC * _NS
_CHUNK = 128
_PER_W = 3840
_NCHUNK = _PER_W // _CHUNK
_BPAD = _NW * _PER_W

_AB = 1000
_EB = _AB * M
_GRID = N // _AB
_XB = 1000
_BAB = 200
_BEB = _BAB * M
_BGRID = N // _BAB
_EMB_B = 1000
IN = 2 * AF + NBR_F


_NBUF = 10
_NGROUP = _NCHUNK // _NBUF


def _sc_gather(table, idx2d):
    mesh = plsc.VectorSubcoreMesh(core_axis_name="c", subcore_axis_name="s")

    scratch = [pltpu.VMEM_SHARED((N, AF), jnp.float32),
               pltpu.VMEM((_NCHUNK, _CHUNK), jnp.int32)]
    scratch += [pltpu.VMEM((_CHUNK, AF), jnp.float32) for _ in range(_NBUF)]
    scratch += [pltpu.SemaphoreType.DMA, pltpu.SemaphoreType.DMA]

    @functools.partial(
        pl.kernel,
        mesh=mesh,
        compiler_params=pltpu.CompilerParams(use_tc_tiling_on_sc=False),
        out_type=jax.ShapeDtypeStruct((_BPAD, AF), jnp.float32),
        scratch_types=scratch,
    )
    def k(table_hbm, idx_hbm, out_hbm, tbl_s, idx_v, *rest):
        bufs = rest[:_NBUF]
        gsem, ssem = rest[_NBUF], rest[_NBUF + 1]
        sid = lax.axis_index("s")
        wid = sid * _NC + lax.axis_index("c")
        base = wid * _PER_W
        pltpu.sync_copy(idx_hbm.at[pl.ds(wid * _NCHUNK, _NCHUNK)], idx_v)

        rows_per = N // 10

        @pl.when(sid < 10)
        def _():
            pltpu.sync_copy(
                table_hbm.at[pl.ds(sid * rows_per, rows_per)],
                tbl_s.at[pl.ds(sid * rows_per, rows_per)])

        plsc.subcore_barrier()

        def body(g, carry):
            c0 = g * _NBUF
            gets = [
                pltpu.async_copy(tbl_s.at[idx_v.at[c0 + b]], bufs[b], gsem)
                for b in range(_NBUF)
            ]
            for cp in gets:
                cp.wait()
            puts = [
                pltpu.async_copy(
                    bufs[b],
                    out_hbm.at[pl.ds(base + (c0 + b) * _CHUNK, _CHUNK)],
                    ssem)
                for b in range(_NBUF)
            ]
            for cp in puts:
                cp.wait()
            return carry

        lax.fori_loop(0, _NGROUP, body, 0)

    return k(table, idx2d)


def _embed_body(af_ref, w_ref, b_ref, o_ref):
    o_ref[...] = (
        jnp.dot(af_ref[...], w_ref[...], preferred_element_type=jnp.float32)
        + b_ref[...]
    )


def _embed(atom_fea, w, b):
    orig = atom_fea.shape[1]
    return pl.pallas_call(
        _embed_body,
        grid=(N // _EMB_B,),
        in_specs=[
            pl.BlockSpec((_EMB_B, orig), lambda i: (i, 0)),
            pl.BlockSpec((orig, AF), lambda i: (0, 0)),
            pl.BlockSpec((1, AF), lambda i: (0, 0)),
        ],
        out_specs=pl.BlockSpec((_EMB_B, AF), lambda i: (i, 0)),
        out_shape=jax.ShapeDtypeStruct((N, AF), jnp.float32),
    )(atom_fea, w, b.reshape(1, AF))


def _edge_gemm(g, nbr3, x, ws, wn, we, b, width):
    u = jnp.dot(x, ws, preferred_element_type=jnp.float32)
    ye = jnp.dot(g, wn, preferred_element_type=jnp.float32)
    ye = ye + jnp.dot(nbr3.reshape(_EB, NBR_F), we,
                      preferred_element_type=jnp.float32)
    return ye.reshape(_AB, M, width) + u[:, None, :] + b[None]


def _conv_fused_body(g_ref, nbr_ref, x_ref, w_ref, b_ref, g1_ref, b1_ref,
                     st_ref, s_ref, st2_ref):
    ph = pl.program_id(0)
    i = pl.program_id(1)
    w = w_ref[0]

    @pl.when(ph == 0)
    def _():
        y = _edge_gemm(g_ref[...], nbr_ref[...], x_ref[...],
                       w[0:AF], w[AF:2 * AF], w[2 * AF:], b_ref[0], F2)
        y2d = y.reshape(_EB, F2)
        ones = jnp.ones((8, _EB), jnp.float32)
        s1 = jnp.dot(ones, y2d, preferred_element_type=jnp.float32)[0:1]
        s2 = jnp.dot(ones, y2d * y2d, preferred_element_type=jnp.float32)[0:1]

        @pl.when(i == 0)
        def _():
            st_ref[...] = jnp.zeros_like(st_ref)

        st_ref[0:1, :] = st_ref[0:1, :] + s1
        st_ref[1:2, :] = st_ref[1:2, :] + s2

    @pl.when(ph == 1)
    def _():
        cnt = jnp.float32(NM)
        mu = st_ref[0:1, :] / cnt
        var = st_ref[1:2, :] / cnt - mu * mu
        scale = g1_ref[0] * lax.rsqrt(var + 1e-5)
        shift = b1_ref[0] - mu * scale
        wn = w * scale
        bn = b_ref[0] * scale + shift
        yn = _edge_gemm(g_ref[...], nbr_ref[...], x_ref[...],
                        wn[0:AF], wn[AF:2 * AF], wn[2 * AF:], bn, F2)
        filt = jax.nn.sigmoid(yn[:, :, :AF])
        pre = yn[:, :, AF:]
        core = jnp.where(pre >= 0, pre, 0.01 * pre)
        s = jnp.sum(filt * core, axis=1)
        s_ref[...] = s
        t1 = jnp.sum(s, axis=0)
        t2 = jnp.sum(s * s, axis=0)

        @pl.when(i == 0)
        def _():
            st2_ref[...] = jnp.zeros_like(st2_ref)

        st2_ref[0:1, :] = st2_ref[0:1, :] + t1[None]
        st2_ref[1:2, :] = st2_ref[1:2, :] + t2[None]


def _conv_fused(g, nbr_fea, x, fc_W, fc_b, bn1_g, bn1_b, layer):
    return pl.pallas_call(
        _conv_fused_body,
        grid=(2, _GRID),
        in_specs=[
            pl.BlockSpec((_EB, AF), lambda ph, i: (i, 0)),
            pl.BlockSpec((_AB, M, NBR_F), lambda ph, i: (i, 0, 0)),
            pl.BlockSpec((_AB, AF), lambda ph, i: (i, 0)),
            pl.BlockSpec((1, IN, F2), lambda ph, i: (layer, 0, 0)),
            pl.BlockSpec((1, 1, F2), lambda ph, i: (layer, 0, 0)),
            pl.BlockSpec((1, 1, F2), lambda ph, i: (layer, 0, 0)),
            pl.BlockSpec((1, 1, F2), lambda ph, i: (layer, 0, 0)),
        ],
        out_specs=[
            pl.BlockSpec((8, F2), lambda ph, i: (0, 0)),
            pl.BlockSpec((_AB, AF), lambda ph, i: (i * ph, 0)),
            pl.BlockSpec((8, AF), lambda ph, i: (0, 0)),
        ],
        out_shape=[
            jax.ShapeDtypeStruct((8, F2), jnp.float32),
            jax.ShapeDtypeStruct((N, AF), jnp.float32),
            jax.ShapeDtypeStruct((8, AF), jnp.float32),
        ],
    )(g, nbr_fea, x, fc_W, fc_b, bn1_g, bn1_b)


def _update_body(x_ref, s_ref, st2_ref, g2_ref, b2_ref, o_ref):
    cnt = jnp.float32(N)
    mu = st2_ref[0:1, :] / cnt
    var = st2_ref[1:2, :] / cnt - mu * mu
    scale = g2_ref[0] * lax.rsqrt(var + 1e-5)
    shift = b2_ref[0] - mu * scale
    t = x_ref[...] + s_ref[...] * scale + shift
    o_ref[...] = jnp.where(t >= 0, t, 0.01 * t)


def _update(x, s, st2, bn2_g, bn2_b, layer):
    return pl.pallas_call(
        _update_body,
        grid=(N // _XB,),
        in_specs=[
            pl.BlockSpec((_XB, AF), lambda i: (i, 0)),
            pl.BlockSpec((_XB, AF), lambda i: (i, 0)),
            pl.BlockSpec((8, AF), lambda i: (0, 0)),
            pl.BlockSpec((1, 1, AF), lambda i: (layer, 0, 0)),
            pl.BlockSpec((1, 1, AF), lambda i: (layer, 0, 0)),
        ],
        out_specs=pl.BlockSpec((_XB, AF), lambda i: (i, 0)),
        out_shape=jax.ShapeDtypeStruct((N, AF), jnp.float32),
    )(x, s, st2, bn2_g, bn2_b)


def _bond_gemm_body(g_ref, nbr_ref, wdc_ref, o_ref):
    wdc = wdc_ref[...]
    ye = jnp.dot(g_ref[...], wdc[AF:2 * AF], preferred_element_type=jnp.float32)
    ye = ye + jnp.dot(nbr_ref[...].reshape(_EB, NBR_F), wdc[2 * AF:],
                      preferred_element_type=jnp.float32)
    o_ref[...] = ye


def _bond_gemm(g, nbr_fea, wdc):
    return pl.pallas_call(
        _bond_gemm_body,
        grid=(_GRID,),
        in_specs=[
            pl.BlockSpec((_EB, AF), lambda i: (i, 0)),
            pl.BlockSpec((_AB, M, NBR_F), lambda i: (i, 0, 0)),
            pl.BlockSpec((IN, 2), lambda i: (0, 0)),
        ],
        out_specs=pl.BlockSpec((_EB, 2), lambda i: (i, 0)),
        out_shape=jax.ShapeDtypeStruct((NM, 2), jnp.float32),
    )(g, nbr_fea, wdc)


def _bond_final_body(yd_ref, yc_ref, x_ref, offp_ref, nposp_ref, apos_ref,
                     aidx_ref, cells_ref, wdc_ref, bias_ref, od_ref, oc_ref):
    u = jnp.dot(x_ref[...], wdc_ref[...][0:AF],
                preferred_element_type=jnp.float32)
    iota = lax.broadcasted_iota(jnp.int32, (_AB, NCRYS), 1)
    oh = (iota == aidx_ref[...]).astype(jnp.float32)
    cell9 = jnp.dot(oh, cells_ref[...], preferred_element_type=jnp.float32)
    offp = offp_ref[...]
    nposp = nposp_ref[...]
    apos = apos_ref[...]
    d2 = None
    for kk in range(3):
        oc = (offp[:, 0:M] * cell9[:, kk:kk + 1]
              + offp[:, M:2 * M] * cell9[:, 3 + kk:4 + kk]
              + offp[:, 2 * M:3 * M] * cell9[:, 6 + kk:7 + kk])
        diff = nposp[:, kk * M:(kk + 1) * M] + oc - apos[:, kk:kk + 1]
        d2 = diff * diff if d2 is None else d2 + diff * diff
    dist = jnp.sqrt(d2 + 1e-12)
    bias = bias_ref[...]
    zd = yd_ref[...] + u[:, 0:1] + bias[0:1, 0:1] + dist
    zc = yc_ref[...] + u[:, 1:2] + bias[0:1, 1:2]
    od_ref[...] = jax.nn.softplus(zd)
    oc_ref[...] = jax.nn.softplus(zc)


def _bond_final(yd, yc, x, offp, nposp, apos, aidx, cells9, wdc, bias):
    return pl.pallas_call(
        _bond_final_body,
        grid=(_GRID,),
        in_specs=[
            pl.BlockSpec((_AB, M), lambda i: (i, 0)),
            pl.BlockSpec((_AB, M), lambda i: (i, 0)),
            pl.BlockSpec((_AB, AF), lambda i: (i, 0)),
            pl.BlockSpec((_AB, 3 * M), lambda i: (i, 0)),
            pl.BlockSpec((_AB, 3 * M), lambda i: (i, 0)),
            pl.BlockSpec((_AB, 3), lambda i: (i, 0)),
            pl.BlockSpec((_AB, 1), lambda i: (i, 0)),
            pl.BlockSpec((NCRYS, 9), lambda i: (0, 0)),
            pl.BlockSpec((IN, 2), lambda i: (0, 0)),
            pl.BlockSpec((1, 2), lambda i: (0, 0)),
        ],
        out_specs=[
            pl.BlockSpec((_AB, M), lambda i: (i, 0)),
            pl.BlockSpec((_AB, M), lambda i: (i, 0)),
        ],
        out_shape=[
            jax.ShapeDtypeStruct((N, M), jnp.float32),
            jax.ShapeDtypeStruct((N, M), jnp.float32),
        ],
    )(yd, yc, x, offp, nposp, apos, aidx, cells9, wdc, bias)


def kernel(atom_fea, nbr_fea, nbr_fea_idx, nbr_fea_offset, crystal_atom_idx,
           atom_pos, nbr_pos, atom_pos_idx, cells, fixed_atom_mask,
           atom_pos_final, W_emb, b_emb, fc_W, fc_b, bn1_g, bn1_b, bn2_g,
           bn2_b, Wd, bd, Wc, bc):
    f32 = jnp.float32
    idx_flat = nbr_fea_idx.astype(jnp.int32).reshape(-1)
    idx_pad = jnp.concatenate(
        [idx_flat, jnp.zeros((_BPAD - NM,), jnp.int32)]
    ).reshape(_NW * _NCHUNK, _CHUNK)
    x = _embed(atom_fea.astype(f32), W_emb.astype(f32), b_emb.astype(f32))
    nbr_fea = nbr_fea.astype(f32)
    fc_W = fc_W.astype(f32)
    fc_b = fc_b.astype(f32).reshape(-1, 1, F2)
    bn1_g = bn1_g.astype(f32).reshape(-1, 1, F2)
    bn1_b = bn1_b.astype(f32).reshape(-1, 1, F2)
    bn2_g = bn2_g.astype(f32).reshape(-1, 1, AF)
    bn2_b = bn2_b.astype(f32).reshape(-1, 1, AF)
    for i in range(fc_W.shape[0]):
        g = _sc_gather(x, idx_pad)
        _, s, st2 = _conv_fused(g, nbr_fea, x, fc_W, fc_b, bn1_g, bn1_b, i)
        x = _update(x, s, st2, bn2_g, bn2_b, i)
    g = _sc_gather(x, idx_pad)
    wdc = jnp.concatenate([Wd.astype(f32), Wc.astype(f32)], axis=1)
    bias = jnp.concatenate([bd.astype(f32), bc.astype(f32) - 4.0]).reshape(1, 2)
    ye = _bond_gemm(g, nbr_fea, wdc)
    yd = ye[:, 0].reshape(N, M)
    yc = ye[:, 1].reshape(N, M)
    offp = nbr_fea_offset.astype(f32).transpose(0, 2, 1).reshape(N, 3 * M)
    nposp = nbr_pos.astype(f32).transpose(0, 2, 1).reshape(N, 3 * M)
    od, oc = _bond_final(yd, yc, x, offp, nposp, atom_pos.astype(f32),
                         atom_pos_idx.astype(jnp.int32).reshape(N, 1),
                         cells.astype(f32).reshape(NCRYS, 9), wdc, bias)
    return jnp.stack([od, oc], axis=2)


# speedup vs baseline: 1.0036x; 1.0036x over previous
import functools

import jax
import jax.numpy as jnp
from jax import lax
from jax.experimental import pallas as pl
from jax.experimental.pallas import tpu as pltpu
from jax.experimental.pallas import tpu_sc as plsc

N = 10000
M = 12
AF = 64
NBR_F = 16
F2 = 2 * AF
NCRYS = 128
NM = N * M

_NC = 2
_NS = 16
_NW = _NC * _NS
_CHUNK = 128
_PER_W = 3840
_NCHUNK = _PER_W // _CHUNK
_BPAD = _NW * _PER_W

_AB = 1000
_EB = _AB * M
_GRID = N // _AB
_XB = 1000
_BAB = 200
_BEB = _BAB * M
_BGRID = N // _BAB
_EMB_B = 1000
IN = 2 * AF + NBR_F


_NBUF = 10
_NGROUP = _NCHUNK // _NBUF


def _sc_gather(table, idx2d):
    mesh = plsc.VectorSubcoreMesh(core_axis_name="c", subcore_axis_name="s")

    scratch = [pltpu.VMEM_SHARED((N, AF), jnp.float32),
               pltpu.VMEM((_NCHUNK, _CHUNK), jnp.int32)]
    scratch += [pltpu.VMEM((_CHUNK, AF), jnp.float32) for _ in range(_NBUF)]
    scratch += [pltpu.SemaphoreType.DMA, pltpu.SemaphoreType.DMA]

    @functools.partial(
        pl.kernel,
        mesh=mesh,
        compiler_params=pltpu.CompilerParams(use_tc_tiling_on_sc=False),
        out_type=jax.ShapeDtypeStruct((_BPAD, AF), jnp.float32),
        scratch_types=scratch,
    )
    def k(table_hbm, idx_hbm, out_hbm, tbl_s, idx_v, *rest):
        bufs = rest[:_NBUF]
        gsem, ssem = rest[_NBUF], rest[_NBUF + 1]
        sid = lax.axis_index("s")
        wid = sid * _NC + lax.axis_index("c")
        base = wid * _PER_W
        pltpu.sync_copy(idx_hbm.at[pl.ds(wid * _NCHUNK, _NCHUNK)], idx_v)

        rows_per = N // 10

        @pl.when(sid < 10)
        def _():
            pltpu.sync_copy(
                table_hbm.at[pl.ds(sid * rows_per, rows_per)],
                tbl_s.at[pl.ds(sid * rows_per, rows_per)])

        plsc.subcore_barrier()

        def body(g, carry):
            c0 = g * _NBUF
            gets = [
                pltpu.async_copy(tbl_s.at[idx_v.at[c0 + b]], bufs[b], gsem)
                for b in range(_NBUF)
            ]
            for cp in gets:
                cp.wait()
            puts = [
                pltpu.async_copy(
                    bufs[b],
                    out_hbm.at[pl.ds(base + (c0 + b) * _CHUNK, _CHUNK)],
                    ssem)
                for b in range(_NBUF)
            ]
            for cp in puts:
                cp.wait()
            return carry

        lax.fori_loop(0, _NGROUP, body, 0)

    return k(table, idx2d)


def _embed_body(af_ref, w_ref, b_ref, o_ref):
    o_ref[...] = (
        jnp.dot(af_ref[...], w_ref[...], preferred_element_type=jnp.float32)
        + b_ref[...]
    )


def _embed(atom_fea, w, b):
    orig = atom_fea.shape[1]
    return pl.pallas_call(
        _embed_body,
        grid=(N // _EMB_B,),
        in_specs=[
            pl.BlockSpec((_EMB_B, orig), lambda i: (i, 0)),
            pl.BlockSpec((orig, AF), lambda i: (0, 0)),
            pl.BlockSpec((1, AF), lambda i: (0, 0)),
        ],
        out_specs=pl.BlockSpec((_EMB_B, AF), lambda i: (i, 0)),
        out_shape=jax.ShapeDtypeStruct((N, AF), jnp.float32),
    )(atom_fea, w, b.reshape(1, AF))


def _edge_gemm(g, nbr3, x, ws, wn, we, b, width):
    u = jnp.dot(x, ws, preferred_element_type=jnp.float32)
    ye = jnp.dot(g, wn, preferred_element_type=jnp.float32)
    ye = ye + jnp.dot(nbr3.reshape(_EB, NBR_F), we,
                      preferred_element_type=jnp.float32)
    return ye.reshape(_AB, M, width) + u[:, None, :] + b[None]


def _conv_fused_body(g_ref, nbr_ref, x_ref, w_ref, b_ref, g1_ref, b1_ref,
                     st_ref, s_ref, st2_ref):
    ph = pl.program_id(0)
    i = pl.program_id(1)
    w = w_ref[0]

    @pl.when(ph == 0)
    def _():
        y = _edge_gemm(g_ref[...], nbr_ref[...], x_ref[...],
                       w[0:AF], w[AF:2 * AF], w[2 * AF:], b_ref[0], F2)
        s1 = jnp.sum(y, axis=(0, 1))
        s2 = jnp.sum(y * y, axis=(0, 1))

        @pl.when(i == 0)
        def _():
            st_ref[...] = jnp.zeros_like(st_ref)

        st_ref[0:1, :] = st_ref[0:1, :] + s1[None]
        st_ref[1:2, :] = st_ref[1:2, :] + s2[None]

    @pl.when(ph == 1)
    def _():
        cnt = jnp.float32(NM)
        mu = st_ref[0:1, :] / cnt
        var = st_ref[1:2, :] / cnt - mu * mu
        scale = g1_ref[0] * lax.rsqrt(var + 1e-5)
        shift = b1_ref[0] - mu * scale
        wn = w * scale
        bn = b_ref[0] * scale + shift
        yn = _edge_gemm(g_ref[...], nbr_ref[...], x_ref[...],
                        wn[0:AF], wn[AF:2 * AF], wn[2 * AF:], bn, F2)
        filt = jax.nn.sigmoid(yn[:, :, :AF])
        pre = yn[:, :, AF:]
        core = jnp.where(pre >= 0, pre, 0.01 * pre)
        s = jnp.sum(filt * core, axis=1)
        s_ref[...] = s
        t1 = jnp.sum(s, axis=0)
        t2 = jnp.sum(s * s, axis=0)

        @pl.when(i == 0)
        def _():
            st2_ref[...] = jnp.zeros_like(st2_ref)

        st2_ref[0:1, :] = st2_ref[0:1, :] + t1[None]
        st2_ref[1:2, :] = st2_ref[1:2, :] + t2[None]


def _conv_fused(g, nbr_fea, x, fc_W, fc_b, bn1_g, bn1_b, layer):
    return pl.pallas_call(
        _conv_fused_body,
        grid=(2, _GRID),
        in_specs=[
            pl.BlockSpec((_EB, AF), lambda ph, i: (i, 0)),
            pl.BlockSpec((_AB, M, NBR_F), lambda ph, i: (i, 0, 0)),
            pl.BlockSpec((_AB, AF), lambda ph, i: (i, 0)),
            pl.BlockSpec((1, IN, F2), lambda ph, i: (layer, 0, 0)),
            pl.BlockSpec((1, 1, F2), lambda ph, i: (layer, 0, 0)),
            pl.BlockSpec((1, 1, F2), lambda ph, i: (layer, 0, 0)),
            pl.BlockSpec((1, 1, F2), lambda ph, i: (layer, 0, 0)),
        ],
        out_specs=[
            pl.BlockSpec((8, F2), lambda ph, i: (0, 0)),
            pl.BlockSpec((_AB, AF), lambda ph, i: (i * ph, 0)),
            pl.BlockSpec((8, AF), lambda ph, i: (0, 0)),
        ],
        out_shape=[
            jax.ShapeDtypeStruct((8, F2), jnp.float32),
            jax.ShapeDtypeStruct((N, AF), jnp.float32),
            jax.ShapeDtypeStruct((8, AF), jnp.float32),
        ],
    )(g, nbr_fea, x, fc_W, fc_b, bn1_g, bn1_b)


def _update_body(x_ref, s_ref, st2_ref, g2_ref, b2_ref, o_ref):
    cnt = jnp.float32(N)
    mu = st2_ref[0:1, :] / cnt
    var = st2_ref[1:2, :] / cnt - mu * mu
    scale = g2_ref[0] * lax.rsqrt(var + 1e-5)
    shift = b2_ref[0] - mu * scale
    t = x_ref[...] + s_ref[...] * scale + shift
    o_ref[...] = jnp.where(t >= 0, t, 0.01 * t)


def _update(x, s, st2, bn2_g, bn2_b, layer):
    return pl.pallas_call(
        _update_body,
        grid=(N // _XB,),
        in_specs=[
            pl.BlockSpec((_XB, AF), lambda i: (i, 0)),
            pl.BlockSpec((_XB, AF), lambda i: (i, 0)),
            pl.BlockSpec((8, AF), lambda i: (0, 0)),
            pl.BlockSpec((1, 1, AF), lambda i: (layer, 0, 0)),
            pl.BlockSpec((1, 1, AF), lambda i: (layer, 0, 0)),
        ],
        out_specs=pl.BlockSpec((_XB, AF), lambda i: (i, 0)),
        out_shape=jax.ShapeDtypeStruct((N, AF), jnp.float32),
    )(x, s, st2, bn2_g, bn2_b)


def _bond_gemm_body(g_ref, nbr_ref, wdc_ref, o_ref):
    wdc = wdc_ref[...]
    ye = jnp.dot(g_ref[...], wdc[AF:2 * AF], preferred_element_type=jnp.float32)
    ye = ye + jnp.dot(nbr_ref[...].reshape(_EB, NBR_F), wdc[2 * AF:],
                      preferred_element_type=jnp.float32)
    o_ref[...] = ye


def _bond_gemm(g, nbr_fea, wdc):
    return pl.pallas_call(
        _bond_gemm_body,
        grid=(_GRID,),
        in_specs=[
            pl.BlockSpec((_EB, AF), lambda i: (i, 0)),
            pl.BlockSpec((_AB, M, NBR_F), lambda i: (i, 0, 0)),
            pl.BlockSpec((IN, 2), lambda i: (0, 0)),
        ],
        out_specs=pl.BlockSpec((_EB, 2), lambda i: (i, 0)),
        out_shape=jax.ShapeDtypeStruct((NM, 2), jnp.float32),
    )(g, nbr_fea, wdc)


def _bond_final_body(yd_ref, yc_ref, x_ref, offp_ref, nposp_ref, apos_ref,
                     aidx_ref, cells_ref, wdc_ref, bias_ref, od_ref, oc_ref):
    u = jnp.dot(x_ref[...], wdc_ref[...][0:AF],
                preferred_element_type=jnp.float32)
    iota = lax.broadcasted_iota(jnp.int32, (_AB, NCRYS), 1)
    oh = (iota == aidx_ref[...]).astype(jnp.float32)
    cell9 = jnp.dot(oh, cells_ref[...], preferred_element_type=jnp.float32)
    offp = offp_ref[...]
    nposp = nposp_ref[...]
    apos = apos_ref[...]
    d2 = None
    for kk in range(3):
        oc = (offp[:, 0:M] * cell9[:, kk:kk + 1]
              + offp[:, M:2 * M] * cell9[:, 3 + kk:4 + kk]
              + offp[:, 2 * M:3 * M] * cell9[:, 6 + kk:7 + kk])
        diff = nposp[:, kk * M:(kk + 1) * M] + oc - apos[:, kk:kk + 1]
        d2 = diff * diff if d2 is None else d2 + diff * diff
    dist = jnp.sqrt(d2 + 1e-12)
    bias = bias_ref[...]
    zd = yd_ref[...] + u[:, 0:1] + bias[0:1, 0:1] + dist
    zc = yc_ref[...] + u[:, 1:2] + bias[0:1, 1:2]
    od_ref[...] = jax.nn.softplus(zd)
    oc_ref[...] = jax.nn.softplus(zc)


def _bond_final(yd, yc, x, offp, nposp, apos, aidx, cells9, wdc, bias):
    return pl.pallas_call(
        _bond_final_body,
        grid=(_GRID,),
        in_specs=[
            pl.BlockSpec((_AB, M), lambda i: (i, 0)),
            pl.BlockSpec((_AB, M), lambda i: (i, 0)),
            pl.BlockSpec((_AB, AF), lambda i: (i, 0)),
            pl.BlockSpec((_AB, 3 * M), lambda i: (i, 0)),
            pl.BlockSpec((_AB, 3 * M), lambda i: (i, 0)),
            pl.BlockSpec((_AB, 3), lambda i: (i, 0)),
            pl.BlockSpec((_AB, 1), lambda i: (i, 0)),
            pl.BlockSpec((NCRYS, 9), lambda i: (0, 0)),
            pl.BlockSpec((IN, 2), lambda i: (0, 0)),
            pl.BlockSpec((1, 2), lambda i: (0, 0)),
        ],
        out_specs=[
            pl.BlockSpec((_AB, M), lambda i: (i, 0)),
            pl.BlockSpec((_AB, M), lambda i: (i, 0)),
        ],
        out_shape=[
            jax.ShapeDtypeStruct((N, M), jnp.float32),
            jax.ShapeDtypeStruct((N, M), jnp.float32),
        ],
    )(yd, yc, x, offp, nposp, apos, aidx, cells9, wdc, bias)


def kernel(atom_fea, nbr_fea, nbr_fea_idx, nbr_fea_offset, crystal_atom_idx,
           atom_pos, nbr_pos, atom_pos_idx, cells, fixed_atom_mask,
           atom_pos_final, W_emb, b_emb, fc_W, fc_b, bn1_g, bn1_b, bn2_g,
           bn2_b, Wd, bd, Wc, bc):
    f32 = jnp.float32
    idx_flat = nbr_fea_idx.astype(jnp.int32).reshape(-1)
    idx_pad = jnp.concatenate(
        [idx_flat, jnp.zeros((_BPAD - NM,), jnp.int32)]
    ).reshape(_NW * _NCHUNK, _CHUNK)
    x = _embed(atom_fea.astype(f32), W_emb.astype(f32), b_emb.astype(f32))
    nbr_fea = nbr_fea.astype(f32)
    fc_W = fc_W.astype(f32)
    fc_b = fc_b.astype(f32).reshape(-1, 1, F2)
    bn1_g = bn1_g.astype(f32).reshape(-1, 1, F2)
    bn1_b = bn1_b.astype(f32).reshape(-1, 1, F2)
    bn2_g = bn2_g.astype(f32).reshape(-1, 1, AF)
    bn2_b = bn2_b.astype(f32).reshape(-1, 1, AF)
    for i in range(fc_W.shape[0]):
        g = _sc_gather(x, idx_pad)
        _, s, st2 = _conv_fused(g, nbr_fea, x, fc_W, fc_b, bn1_g, bn1_b, i)
        x = _update(x, s, st2, bn2_g, bn2_b, i)
    g = _sc_gather(x, idx_pad)
    wdc = jnp.concatenate([Wd.astype(f32), Wc.astype(f32)], axis=1)
    bias = jnp.concatenate([bd.astype(f32), bc.astype(f32) - 4.0]).reshape(1, 2)
    ye = _bond_gemm(g, nbr_fea, wdc)
    yd = ye[:, 0].reshape(N, M)
    yc = ye[:, 1].reshape(N, M)
    offp = nbr_fea_offset.astype(f32).transpose(0, 2, 1).reshape(N, 3 * M)
    nposp = nbr_pos.astype(f32).transpose(0, 2, 1).reshape(N, 3 * M)
    od, oc = _bond_final(yd, yc, x, offp, nposp, atom_pos.astype(f32),
                         atom_pos_idx.astype(jnp.int32).reshape(N, 1),
                         cells.astype(f32).reshape(NCRYS, 9), wdc, bias)
    return jnp.stack([od, oc], axis=2)
